# baseline (speedup 1.0000x reference)
import functools

import jax
import jax.numpy as jnp
from jax import lax
from jax.experimental import pallas as pl
from jax.experimental.pallas import tpu as pltpu
from jax.experimental.pallas import tpu_sc as plsc

D_MODEL = 64
IDX_W = 128
S = 4
CHUNK_ROWS = S * IDX_W
NUM_CORES = 2
NUM_SUBCORES = 16
NUM_WORKERS = NUM_CORES * NUM_SUBCORES


def _gather_body(x_hbm, table_hbm, out_hbm, idx_v, rows_v, gat_sem):
    wid = lax.axis_index("s") * NUM_CORES + lax.axis_index("c")
    n_rows_total = x_hbm.shape[0]
    rows_per_w = n_rows_total // NUM_WORKERS
    n_chunks = rows_per_w // S
    r0 = wid * rows_per_w

    def chunk(g, carry):
        rbase = r0 + g * S
        pltpu.sync_copy(x_hbm.at[pl.ds(rbase, S)], idx_v)
        waits = []
        for j in range(S):
            waits.append(
                pltpu.async_copy(
                    table_hbm.at[idx_v.at[j]],
                    rows_v.at[pl.ds(j * IDX_W, IDX_W)],
                    gat_sem,
                )
            )
        for w in waits:
            w.wait()
        pltpu.sync_copy(rows_v, out_hbm.at[pl.ds(rbase * IDX_W, CHUNK_ROWS)])
        return carry

    lax.fori_loop(0, n_chunks, chunk, 0)


def _make_sc_gather(n_rows):
    mesh = plsc.VectorSubcoreMesh(
        core_axis_name="c",
        subcore_axis_name="s",
        num_cores=NUM_CORES,
        num_subcores=NUM_SUBCORES,
    )
    return pl.kernel(
        _gather_body,
        out_type=jax.ShapeDtypeStruct((n_rows * IDX_W, D_MODEL), jnp.float32),
        mesh=mesh,
        scratch_types=[
            pltpu.VMEM((S, IDX_W), jnp.int32),
            pltpu.VMEM((CHUNK_ROWS, D_MODEL), jnp.float32),
            pltpu.SemaphoreType.DMA,
        ],
        compiler_params=pltpu.CompilerParams(use_tc_tiling_on_sc=False),
    )


@jax.jit
def kernel(x, table):
    b, h = x.shape
    n_flat = b * h
    xf = x.reshape(n_flat // IDX_W, IDX_W).astype(jnp.int32)
    out = _make_sc_gather(n_flat // IDX_W)(xf, table)
    return out.reshape(b, h, D_MODEL)


# speedup vs baseline: 1.0737x; 1.0737x over previous
import functools

import jax
import jax.numpy as jnp
from jax import lax
from jax.experimental import pallas as pl
from jax.experimental.pallas import tpu as pltpu
from jax.experimental.pallas import tpu_sc as plsc

D_MODEL = 64
IDX_W = 128
S = 4
CHUNK_ROWS = S * IDX_W
NUM_CORES = 2
NUM_SUBCORES = 16
NUM_WORKERS = NUM_CORES * NUM_SUBCORES


def _gather_body(x_hbm, table_hbm, out_hbm, idx_v, rows_v,
                 idx_sem, gat_sem, out_sem):
    wid = lax.axis_index("s") * NUM_CORES + lax.axis_index("c")
    n_rows_total = x_hbm.shape[0]
    rows_per_w = n_rows_total // NUM_WORKERS
    n_chunks = rows_per_w // S
    r0 = wid * rows_per_w

    def idx_copy(g, p):
        return pltpu.make_async_copy(
            x_hbm.at[pl.ds(r0 + g * S, S)], idx_v.at[p], idx_sem)

    def out_copy(g, p):
        return pltpu.make_async_copy(
            rows_v.at[p],
            out_hbm.at[pl.ds((r0 + g * S) * IDX_W, CHUNK_ROWS)],
            out_sem)

    idx_copy(0, 0).start()
    idx_copy(1, 1).start()

    def pair(i, carry):
        for p in (0, 1):
            g = 2 * i + p
            idx_copy(g, p).wait()

            @pl.when(i >= 1)
            def _wait_prev_out():
                out_copy(g - 2, p).wait()

            waits = [
                pltpu.async_copy(
                    table_hbm.at[idx_v.at[p].at[j]],
                    rows_v.at[p].at[pl.ds(j * IDX_W, IDX_W)],
                    gat_sem,
                )
                for j in range(S)
            ]
            for w in waits:
                w.wait()
            out_copy(g, p).start()

            @pl.when(g + 2 < n_chunks)
            def _prefetch_idx():
                idx_copy(g + 2, p).start()
        return carry

    lax.fori_loop(0, n_chunks // 2, pair, 0)
    out_copy(n_chunks - 2, 0).wait()
    out_copy(n_chunks - 1, 1).wait()


def _make_sc_gather(n_rows):
    mesh = plsc.VectorSubcoreMesh(
        core_axis_name="c",
        subcore_axis_name="s",
        num_cores=NUM_CORES,
        num_subcores=NUM_SUBCORES,
    )
    return pl.kernel(
        _gather_body,
        out_type=jax.ShapeDtypeStruct((n_rows * IDX_W, D_MODEL), jnp.float32),
        mesh=mesh,
        scratch_types=[
            pltpu.VMEM((2, S, IDX_W), jnp.int32),
            pltpu.VMEM((2, CHUNK_ROWS, D_MODEL), jnp.float32),
            pltpu.SemaphoreType.DMA,
            pltpu.SemaphoreType.DMA,
            pltpu.SemaphoreType.DMA,
        ],
        compiler_params=pltpu.CompilerParams(use_tc_tiling_on_sc=False),
    )


@jax.jit
def kernel(x, table):
    b, h = x.shape
    n_flat = b * h
    xf = x.reshape(n_flat // IDX_W, IDX_W).astype(jnp.int32)
    out = _make_sc_gather(n_flat // IDX_W)(xf, table)
    return out.reshape(b, h, D_MODEL)


# speedup vs baseline: 1.0741x; 1.0004x over previous
import functools

import jax
import jax.numpy as jnp
from jax import lax
from jax.experimental import pallas as pl
from jax.experimental.pallas import tpu as pltpu
from jax.experimental.pallas import tpu_sc as plsc

D_MODEL = 64
IDX_W = 128
S = 4
CHUNK_ROWS = S * IDX_W
NUM_CORES = 2
NUM_SUBCORES = 16
NUM_WORKERS = NUM_CORES * NUM_SUBCORES


def _gather_body(x_hbm, table_hbm, out_hbm, idx_v, rows_v,
                 idx_sem, gat_sem, out_sem):
    wid = lax.axis_index("s") * NUM_CORES + lax.axis_index("c")
    n_rows_total = x_hbm.shape[0]
    rows_per_w = n_rows_total // NUM_WORKERS
    n_chunks = rows_per_w // S
    r0 = wid * rows_per_w

    def idx_copy(g, p):
        return pltpu.make_async_copy(
            x_hbm.at[pl.ds(r0 + g * S, S)], idx_v.at[p], idx_sem)

    def gat_copy(p, j):
        return pltpu.make_async_copy(
            table_hbm.at[idx_v.at[p].at[j]],
            rows_v.at[p].at[pl.ds(j * IDX_W, IDX_W)],
            gat_sem)

    def out_copy(g, p):
        return pltpu.make_async_copy(
            rows_v.at[p],
            out_hbm.at[pl.ds((r0 + g * S) * IDX_W, CHUNK_ROWS)],
            out_sem)

    idx_copy(0, 0).start()
    idx_copy(0, 0).wait()
    for j in range(S):
        gat_copy(0, j).start()
    idx_copy(1, 1).start()

    def pair(i, carry):
        for p in (0, 1):
            g = 2 * i + p
            q = 1 - p
            for j in range(S):
                gat_copy(p, j).wait()
            out_copy(g, p).start()

            @pl.when(g >= 1)
            def _wait_prev_out():
                out_copy(g - 1, q).wait()

            @pl.when(g + 1 < n_chunks)
            def _fire_next_gathers():
                idx_copy(g + 1, q).wait()
                for j in range(S):
                    gat_copy(q, j).start()

            @pl.when(g + 2 < n_chunks)
            def _prefetch_idx():
                idx_copy(g + 2, p).start()
        return carry

    lax.fori_loop(0, n_chunks // 2, pair, 0)
    out_copy(n_chunks - 1, 1).wait()


def _make_sc_gather(n_rows):
    mesh = plsc.VectorSubcoreMesh(
        core_axis_name="c",
        subcore_axis_name="s",
        num_cores=NUM_CORES,
        num_subcores=NUM_SUBCORES,
    )
    return pl.kernel(
        _gather_body,
        out_type=jax.ShapeDtypeStruct((n_rows * IDX_W, D_MODEL), jnp.float32),
        mesh=mesh,
        scratch_types=[
            pltpu.VMEM((2, S, IDX_W), jnp.int32),
            pltpu.VMEM((2, CHUNK_ROWS, D_MODEL), jnp.float32),
            pltpu.SemaphoreType.DMA,
            pltpu.SemaphoreType.DMA,
            pltpu.SemaphoreType.DMA,
        ],
        compiler_params=pltpu.CompilerParams(use_tc_tiling_on_sc=False),
    )


@jax.jit
def kernel(x, table):
    b, h = x.shape
    n_flat = b * h
    xf = x.reshape(n_flat // IDX_W, IDX_W).astype(jnp.int32)
    out = _make_sc_gather(n_flat // IDX_W)(xf, table)
    return out.reshape(b, h, D_MODEL)
